# baseline (speedup 1.0000x reference)
import functools

import jax
import jax.numpy as jnp
from jax import lax
from jax.experimental import pallas as pl
from jax.experimental.pallas import tpu as pltpu
from jax.experimental.pallas import tpu_sc as plsc

NC = 2
NS = 16
CHUNK = 80


def _mesh():
    return plsc.VectorSubcoreMesh(core_axis_name="c", subcore_axis_name="s")


def _sc_degrees(src, dst, zeros1d):
    e = src.shape[0]
    n = zeros1d.shape[0]
    per_tile = e // (NC * NS)
    steps = per_tile // CHUNK

    @functools.partial(
        pl.kernel,
        mesh=_mesh(),
        compiler_params=pltpu.CompilerParams(needs_layout_passes=False),
        out_type=(
            jax.ShapeDtypeStruct((NC * NS * n,), jnp.float32),
            jax.ShapeDtypeStruct((NC * NS * n,), jnp.float32),
        ),
        scratch_types=[
            pltpu.VMEM((e // (NC * NS),), jnp.int32),
            pltpu.VMEM((e // (NC * NS),), jnp.int32),
            pltpu.VMEM((n,), jnp.float32),
            pltpu.VMEM((n,), jnp.float32),
        ],
    )
    def deg_kernel(src_hbm, dst_hbm, z_hbm, outa_hbm, outb_hbm,
                   idx_s, idx_d, ha, hb):
        cid = lax.axis_index("c")
        sid = lax.axis_index("s")
        wid = cid * NS + sid
        pltpu.sync_copy(z_hbm, ha)
        pltpu.sync_copy(z_hbm, hb)
        base = wid * per_tile
        ones = jnp.full((16,), 1.0, jnp.float32)

        pltpu.sync_copy(src_hbm.at[pl.ds(base, per_tile)], idx_s)
        pltpu.sync_copy(dst_hbm.at[pl.ds(base, per_tile)], idx_d)

        def body(j, carry):
            plsc.addupdate_scatter(ha, [idx_s[pl.ds(j * 16, 16)]], ones)
            plsc.addupdate_scatter(hb, [idx_d[pl.ds(j * 16, 16)]], ones)
            return carry

        lax.fori_loop(0, per_tile // 16, body, 0)
        pltpu.sync_copy(ha, outa_hbm.at[pl.ds(wid * n, n)])
        pltpu.sync_copy(hb, outb_hbm.at[pl.ds(wid * n, n)])

    return deg_kernel(src, dst, zeros1d)


def _sc_gather_scatter(table, src_flat, dst3, zeros128):
    nw, steps, chunk = dst3.shape
    pt = steps * chunk
    n, d = zeros128.shape
    s0 = (n // (8 * NS)) * 8
    tail = n - NS * s0
    assert steps % 2 == 0

    @functools.partial(
        pl.kernel,
        mesh=_mesh(),
        out_type=jax.ShapeDtypeStruct((NC, n, d), jnp.float32),
        scratch_types=[
            pltpu.VMEM((pt,), jnp.int32),
            pltpu.VMEM((steps, chunk), jnp.int32),
            pltpu.VMEM((chunk, d), jnp.float32),
            pltpu.VMEM((chunk, d), jnp.float32),
            pltpu.SemaphoreType.DMA,
            pltpu.SemaphoreType.DMA,
            pltpu.VMEM_SHARED((n, d), jnp.float32),
        ],
    )
    def gs_kernel(tab_hbm, src_hbm, dst_hbm, z_hbm, out_hbm,
                  idx_s, idx_d, r0, r1, sem0, sem1, acc):
        cid = lax.axis_index("c")
        sid = lax.axis_index("s")
        wid = cid * NS + sid

        def each_stripe(fn):
            fn(pl.ds(sid * s0, s0))
            if tail:
                @pl.when(sid == NS - 1)
                def _():
                    fn(pl.ds(NS * s0, tail))

        each_stripe(lambda st: pltpu.sync_copy(z_hbm.at[st], acc.at[st]))
        pltpu.sync_copy(src_hbm.at[pl.ds(wid * pt, pt)], idx_s)
        pltpu.sync_copy(dst_hbm.at[wid], idx_d)
        plsc.subcore_barrier()

        def start(j, rb, sem):
            pltpu.async_copy(tab_hbm.at[idx_s.at[pl.ds(j * chunk, chunk)]], rb, sem)

        def drain(j, rb, sem):
            pltpu.make_async_copy(
                tab_hbm.at[idx_s.at[pl.ds(j * chunk, chunk)]], rb, sem).wait()

        start(0, r0, sem0)

        def body(jj, carry):
            j0 = 2 * jj
            j1 = j0 + 1
            start(j1, r1, sem1)
            drain(j0, r0, sem0)
            pltpu.sync_copy(r0, acc.at[idx_d.at[j0]], add=True)
            start(j0 + 2, r0, sem0)
            drain(j1, r1, sem1)
            pltpu.sync_copy(r1, acc.at[idx_d.at[j1]], add=True)
            return carry

        lax.fori_loop(0, steps // 2 - 1, body, 0)
        start(steps - 1, r1, sem1)
        drain(steps - 2, r0, sem0)
        pltpu.sync_copy(r0, acc.at[idx_d.at[steps - 2]], add=True)
        drain(steps - 1, r1, sem1)
        pltpu.sync_copy(r1, acc.at[idx_d.at[steps - 1]], add=True)

        plsc.subcore_barrier()
        each_stripe(
            lambda st: pltpu.sync_copy(acc.at[st], out_hbm.at[cid].at[st]))

    return gs_kernel(table, src_flat, dst3, zeros128)


def _sc_edge_dots(hn, src_flat, dst_flat, steps, chunk):
    pt = steps * chunk
    e_pad = NC * NS * pt
    n, d = hn.shape
    nsub = d // 16
    rows_per_chunk = chunk // 8
    assert steps % 2 == 0 and chunk % 16 == 0

    @functools.partial(
        pl.kernel,
        mesh=_mesh(),
        compiler_params=pltpu.CompilerParams(needs_layout_passes=False),
        out_type=jax.ShapeDtypeStruct((e_pad // 8, 128), jnp.float32),
        scratch_types=[
            pltpu.VMEM((pt,), jnp.int32),
            pltpu.VMEM((pt,), jnp.int32),
            pltpu.VMEM((chunk, d), jnp.float32),
            pltpu.VMEM((chunk, d), jnp.float32),
            pltpu.VMEM((chunk, d), jnp.float32),
            pltpu.VMEM((chunk, d), jnp.float32),
            pltpu.VMEM((2 * rows_per_chunk, 128), jnp.float32),
            pltpu.SemaphoreType.DMA,
            pltpu.SemaphoreType.DMA,
        ],
    )
    def dot_kernel(hn_hbm, src_hbm, dst_hbm, out_hbm,
                   idx_s, idx_d, s0b, d0b, s1b, d1b, res2, sem0, sem1):
        cid = lax.axis_index("c")
        sid = lax.axis_index("s")
        wid = cid * NS + sid
        base = wid * pt
        row_base = wid * (pt // 8)
        pltpu.sync_copy(src_hbm.at[pl.ds(base, pt)], idx_s)
        pltpu.sync_copy(dst_hbm.at[pl.ds(base, pt)], idx_d)

        def start(j, sb, db, sem):
            sl = pl.ds(j * chunk, chunk)
            pltpu.async_copy(hn_hbm.at[idx_s.at[sl]], sb, sem)
            pltpu.async_copy(hn_hbm.at[idx_d.at[sl]], db, sem)

        def drain(j, sb, db, sem):
            sl = pl.ds(j * chunk, chunk)
            pltpu.make_async_copy(hn_hbm.at[idx_s.at[sl]], sb, sem).wait()
            pltpu.make_async_copy(hn_hbm.at[idx_d.at[sl]], db, sem).wait()

        def compute(parity, sb, db):
            def block(g8, c2):
                row = parity * rows_per_chunk + g8
                for u in range(8):
                    ei = g8 * 8 + u
                    ta = sb[ei, pl.ds(0, 16)] * db[ei, pl.ds(0, 16)]
                    tb = sb[ei, pl.ds(16, 16)] * db[ei, pl.ds(16, 16)]
                    for k in range(2, nsub, 2):
                        ta = ta + sb[ei, pl.ds(16 * k, 16)] * db[ei, pl.ds(16 * k, 16)]
                        tb = tb + sb[ei, pl.ds(16 * (k + 1), 16)] * db[ei, pl.ds(16 * (k + 1), 16)]
                    res2[row, pl.ds(u * 16, 16)] = ta + tb
                return c2

            lax.fori_loop(0, rows_per_chunk, block, 0)

        def flush(jj):
            pltpu.sync_copy(
                res2,
                out_hbm.at[pl.ds(row_base + jj * 2 * rows_per_chunk,
                                 2 * rows_per_chunk)])

        start(0, s0b, d0b, sem0)

        def body(jj, carry):
            j0 = 2 * jj
            j1 = j0 + 1
            start(j1, s1b, d1b, sem1)
            drain(j0, s0b, d0b, sem0)
            compute(0, s0b, d0b)
            start(j0 + 2, s0b, d0b, sem0)
            drain(j1, s1b, d1b, sem1)
            compute(1, s1b, d1b)
            flush(jj)
            return carry

        lax.fori_loop(0, steps // 2 - 1, body, 0)
        start(steps - 1, s1b, d1b, sem1)
        drain(steps - 2, s0b, d0b, sem0)
        compute(0, s0b, d0b)
        drain(steps - 1, s1b, d1b, sem1)
        compute(1, s1b, d1b)
        flush(steps // 2 - 1)

    return dot_kernel(hn, src_flat, dst_flat)


def _tc_norms(hs, hd):
    def f(hs_ref, hd_ref, ns_ref, nd_ref):
        ds_ = jnp.sum(hs_ref[...], axis=0, keepdims=True)
        dd_ = jnp.sum(hd_ref[...], axis=0, keepdims=True)
        ns_ref[...] = lax.rsqrt(jnp.maximum(ds_, 1.0))
        nd_ref[...] = lax.rsqrt(jnp.maximum(dd_, 1.0))

    n = hs.shape[1]
    return pl.pallas_call(
        f,
        out_shape=(
            jax.ShapeDtypeStruct((1, n), jnp.float32),
            jax.ShapeDtypeStruct((1, n), jnp.float32),
        ),
    )(hs, hd)


def _tc_scale_src(x, ns_col):
    def f(x_ref, s_ref, o_ref):
        o_ref[...] = x_ref[...] * s_ref[...]

    return pl.pallas_call(
        f, out_shape=jax.ShapeDtypeStruct(x.shape, jnp.float32)
    )(x, ns_col)


def _tc_combine_mid(p0, p1, nd_col, ns_col, W, b):
    def f(p0_ref, p1_ref, nd_ref, ns_ref, w_ref, b_ref, o_ref):
        agg = (p0_ref[...] + p1_ref[...]) * nd_ref[...]
        h = jnp.dot(agg, w_ref[...], preferred_element_type=jnp.float32) + b_ref[...]
        o_ref[...] = h * ns_ref[...]

    return pl.pallas_call(
        f, out_shape=jax.ShapeDtypeStruct(p0.shape, jnp.float32)
    )(p0, p1, nd_col, ns_col, W, b)


def _tc_combine_final(q0, q1, nd_col, W, b):
    def f(q0_ref, q1_ref, nd_ref, w_ref, b_ref, o_ref):
        agg = (q0_ref[...] + q1_ref[...]) * nd_ref[...]
        h = jnp.dot(agg, w_ref[...], preferred_element_type=jnp.float32) + b_ref[...]
        nrm = jnp.sqrt(jnp.sum(h * h, axis=1, keepdims=True))
        o_ref[...] = h / jnp.maximum(nrm, 1e-8)

    return pl.pallas_call(
        f, out_shape=jax.ShapeDtypeStruct(q0.shape, jnp.float32)
    )(q0, q1, nd_col, W, b)


def _tc_pack_reduce(part_rows, sel):
    def f(p_ref, s_ref, o_ref):
        o_ref[...] = jnp.dot(p_ref[...], s_ref[...],
                             preferred_element_type=jnp.float32)

    r = part_rows.shape[0]
    return pl.pallas_call(
        f, out_shape=jax.ShapeDtypeStruct((r, 8), jnp.float32)
    )(part_rows, sel)


def kernel(x, edge_index, W1, b1, W2, b2):
    n, d = x.shape
    src = edge_index[0].astype(jnp.int32)
    dst = edge_index[1].astype(jnp.int32)
    e = src.shape[0]
    nw = NC * NS
    pt = e // nw
    ch = 96
    steps = -(-pt // ch)
    if steps % 2:
        steps += 1
    pt_pad = steps * ch
    pad = pt_pad - pt
    src2 = src.reshape(nw, pt)
    dst2 = dst.reshape(nw, pt)
    src_flat = jnp.pad(src2, ((0, 0), (0, pad)), constant_values=n).reshape(-1)
    dst_flat = jnp.pad(dst2, ((0, 0), (0, pad))).reshape(-1)
    dst3 = jnp.pad(dst2, ((0, 0), (0, pad))).reshape(nw, steps, ch)

    zeros1d = jnp.zeros((n,), jnp.float32)
    zeros128 = jnp.zeros((n, d), jnp.float32)
    b1r = b1.reshape(1, d)
    b2r = b2.reshape(1, d)

    hs_flat, hd_flat = _sc_degrees(src, dst, zeros1d)
    ns_row, nd_row = _tc_norms(
        hs_flat.reshape(NC * NS, n), hd_flat.reshape(NC * NS, n))
    ns_col = ns_row.reshape(n, 1)
    nd_col = nd_row.reshape(n, 1)

    xs = _tc_scale_src(x, ns_col)

    rowpad = ((0, 8), (0, 0))
    p = _sc_gather_scatter(jnp.pad(xs, rowpad), src_flat, dst3, zeros128)
    h1s = _tc_combine_mid(p[0], p[1], nd_col, ns_col, W1, b1r)

    q = _sc_gather_scatter(jnp.pad(h1s, rowpad), src_flat, dst3, zeros128)
    hn = _tc_combine_final(q[0], q[1], nd_col, W2, b2r)

    part_rows = _sc_edge_dots(jnp.pad(hn, rowpad), src_flat, dst_flat, steps, ch)
    sel = (jnp.arange(128)[:, None] // 16 == jnp.arange(8)[None, :]).astype(
        jnp.float32)
    dots8 = _tc_pack_reduce(part_rows, sel)
    return dots8.reshape(nw, pt_pad)[:, :pt].reshape(-1)


# speedup vs baseline: 1.3077x; 1.3077x over previous
import functools

import jax
import jax.numpy as jnp
from jax import lax
from jax.experimental import pallas as pl
from jax.experimental.pallas import tpu as pltpu
from jax.experimental.pallas import tpu_sc as plsc

NC = 2
NS = 16
CHUNK = 80


def _mesh():
    return plsc.VectorSubcoreMesh(core_axis_name="c", subcore_axis_name="s")


def _sc_degrees(src, dst, zeros1d):
    e = src.shape[0]
    n = zeros1d.shape[0]
    per_tile = e // (NC * NS)
    steps = per_tile // CHUNK

    @functools.partial(
        pl.kernel,
        mesh=_mesh(),
        compiler_params=pltpu.CompilerParams(needs_layout_passes=False),
        out_type=(
            jax.ShapeDtypeStruct((NC * NS * n,), jnp.float32),
            jax.ShapeDtypeStruct((NC * NS * n,), jnp.float32),
        ),
        scratch_types=[
            pltpu.VMEM((e // (NC * NS),), jnp.int32),
            pltpu.VMEM((e // (NC * NS),), jnp.int32),
            pltpu.VMEM((n,), jnp.float32),
            pltpu.VMEM((n,), jnp.float32),
        ],
    )
    def deg_kernel(src_hbm, dst_hbm, z_hbm, outa_hbm, outb_hbm,
                   idx_s, idx_d, ha, hb):
        cid = lax.axis_index("c")
        sid = lax.axis_index("s")
        wid = cid * NS + sid
        pltpu.sync_copy(z_hbm, ha)
        pltpu.sync_copy(z_hbm, hb)
        base = wid * per_tile
        ones = jnp.full((16,), 1.0, jnp.float32)

        pltpu.sync_copy(src_hbm.at[pl.ds(base, per_tile)], idx_s)
        pltpu.sync_copy(dst_hbm.at[pl.ds(base, per_tile)], idx_d)

        def body(j, carry):
            plsc.addupdate_scatter(ha, [idx_s[pl.ds(j * 16, 16)]], ones)
            plsc.addupdate_scatter(hb, [idx_d[pl.ds(j * 16, 16)]], ones)
            return carry

        lax.fori_loop(0, per_tile // 16, body, 0)
        pltpu.sync_copy(ha, outa_hbm.at[pl.ds(wid * n, n)])
        pltpu.sync_copy(hb, outb_hbm.at[pl.ds(wid * n, n)])

    return deg_kernel(src, dst, zeros1d)


def _sc_gather_scatter(table, src_flat, dst3, zeros128):
    nw, steps, chunk = dst3.shape
    pt = steps * chunk
    n, d = zeros128.shape
    s0 = (n // (8 * NS)) * 8
    tail = n - NS * s0
    assert steps % 2 == 0

    @functools.partial(
        pl.kernel,
        mesh=_mesh(),
        out_type=jax.ShapeDtypeStruct((NC, n, d), jnp.float32),
        scratch_types=[
            pltpu.VMEM((pt,), jnp.int32),
            pltpu.VMEM((steps, chunk), jnp.int32),
            pltpu.VMEM((chunk, d), jnp.float32),
            pltpu.VMEM((chunk, d), jnp.float32),
            pltpu.SemaphoreType.DMA,
            pltpu.SemaphoreType.DMA,
            pltpu.VMEM_SHARED((n, d), jnp.float32),
        ],
    )
    def gs_kernel(tab_hbm, src_hbm, dst_hbm, z_hbm, out_hbm,
                  idx_s, idx_d, r0, r1, sem0, sem1, acc):
        cid = lax.axis_index("c")
        sid = lax.axis_index("s")
        wid = cid * NS + sid

        def each_stripe(fn):
            fn(pl.ds(sid * s0, s0))
            if tail:
                @pl.when(sid == NS - 1)
                def _():
                    fn(pl.ds(NS * s0, tail))

        each_stripe(lambda st: pltpu.sync_copy(z_hbm.at[st], acc.at[st]))
        pltpu.sync_copy(src_hbm.at[pl.ds(wid * pt, pt)], idx_s)
        pltpu.sync_copy(dst_hbm.at[wid], idx_d)
        plsc.subcore_barrier()

        def start(j, rb, sem):
            pltpu.async_copy(tab_hbm.at[idx_s.at[pl.ds(j * chunk, chunk)]], rb, sem)

        def drain(j, rb, sem):
            pltpu.make_async_copy(
                tab_hbm.at[idx_s.at[pl.ds(j * chunk, chunk)]], rb, sem).wait()

        start(0, r0, sem0)

        def body(jj, carry):
            j0 = 2 * jj
            j1 = j0 + 1
            start(j1, r1, sem1)
            drain(j0, r0, sem0)
            pltpu.sync_copy(r0, acc.at[idx_d.at[j0]], add=True)
            start(j0 + 2, r0, sem0)
            drain(j1, r1, sem1)
            pltpu.sync_copy(r1, acc.at[idx_d.at[j1]], add=True)
            return carry

        lax.fori_loop(0, steps // 2 - 1, body, 0)
        start(steps - 1, r1, sem1)
        drain(steps - 2, r0, sem0)
        pltpu.sync_copy(r0, acc.at[idx_d.at[steps - 2]], add=True)
        drain(steps - 1, r1, sem1)
        pltpu.sync_copy(r1, acc.at[idx_d.at[steps - 1]], add=True)

        plsc.subcore_barrier()
        each_stripe(
            lambda st: pltpu.sync_copy(acc.at[st], out_hbm.at[cid].at[st]))

    return gs_kernel(table, src_flat, dst3, zeros128)


def _sc_edge_dots(hn, src_flat, dst_flat, steps, chunk):
    pt = steps * chunk
    e_pad = NC * NS * pt
    n8, d = hn.shape
    nsub = d // 16
    rows_per_chunk = chunk // 8
    s0s = (n8 // (8 * NS)) * 8
    tail = n8 - NS * s0s
    assert steps % 2 == 0 and chunk % 16 == 0

    @functools.partial(
        pl.kernel,
        mesh=_mesh(),
        compiler_params=pltpu.CompilerParams(needs_layout_passes=False),
        out_type=jax.ShapeDtypeStruct((e_pad // 8, 128), jnp.float32),
        scratch_types=[
            pltpu.VMEM((pt,), jnp.int32),
            pltpu.VMEM((pt,), jnp.int32),
            pltpu.VMEM((chunk, d), jnp.float32),
            pltpu.VMEM((chunk, d), jnp.float32),
            pltpu.VMEM((chunk, d), jnp.float32),
            pltpu.VMEM((chunk, d), jnp.float32),
            pltpu.VMEM((2 * rows_per_chunk, 128), jnp.float32),
            pltpu.SemaphoreType.DMA,
            pltpu.SemaphoreType.DMA,
            pltpu.VMEM_SHARED((n8, d), jnp.float32),
        ],
    )
    def dot_kernel(hn_hbm, src_hbm, dst_hbm, out_hbm,
                   idx_s, idx_d, s0b, d0b, s1b, d1b, res2, sem0, sem1, tab_s):
        cid = lax.axis_index("c")
        sid = lax.axis_index("s")
        wid = cid * NS + sid
        base = wid * pt
        row_base = wid * (pt // 8)
        st = pl.ds(sid * s0s, s0s)
        pltpu.sync_copy(hn_hbm.at[st], tab_s.at[st])
        if tail:
            @pl.when(sid == NS - 1)
            def _():
                st2 = pl.ds(NS * s0s, tail)
                pltpu.sync_copy(hn_hbm.at[st2], tab_s.at[st2])
        pltpu.sync_copy(src_hbm.at[pl.ds(base, pt)], idx_s)
        pltpu.sync_copy(dst_hbm.at[pl.ds(base, pt)], idx_d)
        plsc.subcore_barrier()

        def start(j, sb, db, sem):
            sl = pl.ds(j * chunk, chunk)
            pltpu.async_copy(tab_s.at[idx_s.at[sl]], sb, sem)
            pltpu.async_copy(tab_s.at[idx_d.at[sl]], db, sem)

        def drain(j, sb, db, sem):
            sl = pl.ds(j * chunk, chunk)
            pltpu.make_async_copy(tab_s.at[idx_s.at[sl]], sb, sem).wait()
            pltpu.make_async_copy(tab_s.at[idx_d.at[sl]], db, sem).wait()

        def compute(parity, sb, db):
            def block(g8, c2):
                row = parity * rows_per_chunk + g8
                for u in range(8):
                    ei = g8 * 8 + u
                    ta = sb[ei, pl.ds(0, 16)] * db[ei, pl.ds(0, 16)]
                    tb = sb[ei, pl.ds(16, 16)] * db[ei, pl.ds(16, 16)]
                    for k in range(2, nsub, 2):
                        ta = ta + sb[ei, pl.ds(16 * k, 16)] * db[ei, pl.ds(16 * k, 16)]
                        tb = tb + sb[ei, pl.ds(16 * (k + 1), 16)] * db[ei, pl.ds(16 * (k + 1), 16)]
                    res2[row, pl.ds(u * 16, 16)] = ta + tb
                return c2

            lax.fori_loop(0, rows_per_chunk, block, 0)

        def flush(jj):
            pltpu.sync_copy(
                res2,
                out_hbm.at[pl.ds(row_base + jj * 2 * rows_per_chunk,
                                 2 * rows_per_chunk)])

        start(0, s0b, d0b, sem0)

        def body(jj, carry):
            j0 = 2 * jj
            j1 = j0 + 1
            start(j1, s1b, d1b, sem1)
            drain(j0, s0b, d0b, sem0)
            compute(0, s0b, d0b)
            start(j0 + 2, s0b, d0b, sem0)
            drain(j1, s1b, d1b, sem1)
            compute(1, s1b, d1b)
            flush(jj)
            return carry

        lax.fori_loop(0, steps // 2 - 1, body, 0)
        start(steps - 1, s1b, d1b, sem1)
        drain(steps - 2, s0b, d0b, sem0)
        compute(0, s0b, d0b)
        drain(steps - 1, s1b, d1b, sem1)
        compute(1, s1b, d1b)
        flush(steps // 2 - 1)

    return dot_kernel(hn, src_flat, dst_flat)


def _tc_norms(hs, hd):
    def f(hs_ref, hd_ref, ns_ref, nd_ref):
        ds_ = jnp.sum(hs_ref[...], axis=0, keepdims=True)
        dd_ = jnp.sum(hd_ref[...], axis=0, keepdims=True)
        ns_ref[...] = lax.rsqrt(jnp.maximum(ds_, 1.0))
        nd_ref[...] = lax.rsqrt(jnp.maximum(dd_, 1.0))

    n = hs.shape[1]
    return pl.pallas_call(
        f,
        out_shape=(
            jax.ShapeDtypeStruct((1, n), jnp.float32),
            jax.ShapeDtypeStruct((1, n), jnp.float32),
        ),
    )(hs, hd)


def _tc_scale_src(x, ns_col):
    def f(x_ref, s_ref, o_ref):
        o_ref[...] = x_ref[...] * s_ref[...]

    return pl.pallas_call(
        f, out_shape=jax.ShapeDtypeStruct(x.shape, jnp.float32)
    )(x, ns_col)


def _tc_combine_mid(p0, p1, nd_col, ns_col, W, b):
    def f(p0_ref, p1_ref, nd_ref, ns_ref, w_ref, b_ref, o_ref):
        agg = (p0_ref[...] + p1_ref[...]) * nd_ref[...]
        h = jnp.dot(agg, w_ref[...], preferred_element_type=jnp.float32) + b_ref[...]
        o_ref[...] = h * ns_ref[...]

    return pl.pallas_call(
        f, out_shape=jax.ShapeDtypeStruct(p0.shape, jnp.float32)
    )(p0, p1, nd_col, ns_col, W, b)


def _tc_combine_final(q0, q1, nd_col, W, b):
    def f(q0_ref, q1_ref, nd_ref, w_ref, b_ref, o_ref):
        agg = (q0_ref[...] + q1_ref[...]) * nd_ref[...]
        h = jnp.dot(agg, w_ref[...], preferred_element_type=jnp.float32) + b_ref[...]
        nrm = jnp.sqrt(jnp.sum(h * h, axis=1, keepdims=True))
        o_ref[...] = h / jnp.maximum(nrm, 1e-8)

    return pl.pallas_call(
        f, out_shape=jax.ShapeDtypeStruct(q0.shape, jnp.float32)
    )(q0, q1, nd_col, W, b)


def _tc_pack_reduce(part_rows, sel):
    def f(p_ref, s_ref, o_ref):
        o_ref[...] = jnp.dot(p_ref[...], s_ref[...],
                             preferred_element_type=jnp.float32)

    r = part_rows.shape[0]
    return pl.pallas_call(
        f, out_shape=jax.ShapeDtypeStruct((r, 8), jnp.float32)
    )(part_rows, sel)


def kernel(x, edge_index, W1, b1, W2, b2):
    n, d = x.shape
    src = edge_index[0].astype(jnp.int32)
    dst = edge_index[1].astype(jnp.int32)
    e = src.shape[0]
    nw = NC * NS
    pt = e // nw
    ch = 96
    steps = -(-pt // ch)
    if steps % 2:
        steps += 1
    pt_pad = steps * ch
    pad = pt_pad - pt
    src2 = src.reshape(nw, pt)
    dst2 = dst.reshape(nw, pt)
    src_flat = jnp.pad(src2, ((0, 0), (0, pad)), constant_values=n).reshape(-1)
    dst_flat = jnp.pad(dst2, ((0, 0), (0, pad))).reshape(-1)
    dst3 = jnp.pad(dst2, ((0, 0), (0, pad))).reshape(nw, steps, ch)

    zeros1d = jnp.zeros((n,), jnp.float32)
    zeros128 = jnp.zeros((n, d), jnp.float32)
    b1r = b1.reshape(1, d)
    b2r = b2.reshape(1, d)

    hs_flat, hd_flat = _sc_degrees(src, dst, zeros1d)
    ns_row, nd_row = _tc_norms(
        hs_flat.reshape(NC * NS, n), hd_flat.reshape(NC * NS, n))
    ns_col = ns_row.reshape(n, 1)
    nd_col = nd_row.reshape(n, 1)

    xs = _tc_scale_src(x, ns_col)

    rowpad = ((0, 8), (0, 0))
    p = _sc_gather_scatter(jnp.pad(xs, rowpad), src_flat, dst3, zeros128)
    h1s = _tc_combine_mid(p[0], p[1], nd_col, ns_col, W1, b1r)

    q = _sc_gather_scatter(jnp.pad(h1s, rowpad), src_flat, dst3, zeros128)
    hn = _tc_combine_final(q[0], q[1], nd_col, W2, b2r)

    ch_d = 32
    steps_d = -(-pt // ch_d)
    if steps_d % 2:
        steps_d += 1
    ptd_pad = steps_d * ch_d
    pad_d = ptd_pad - pt
    srcd_flat = jnp.pad(src2, ((0, 0), (0, pad_d))).reshape(-1)
    dstd_flat = jnp.pad(dst2, ((0, 0), (0, pad_d))).reshape(-1)
    part_rows = _sc_edge_dots(
        jnp.pad(hn, rowpad), srcd_flat, dstd_flat, steps_d, ch_d)
    sel = (jnp.arange(128)[:, None] // 16 == jnp.arange(8)[None, :]).astype(
        jnp.float32)
    dots8 = _tc_pack_reduce(part_rows, sel)
    return dots8.reshape(nw, ptd_pad)[:, :pt].reshape(-1)


# speedup vs baseline: 1.3222x; 1.0111x over previous
import functools

import jax
import jax.numpy as jnp
from jax import lax
from jax.experimental import pallas as pl
from jax.experimental.pallas import tpu as pltpu
from jax.experimental.pallas import tpu_sc as plsc

NC = 2
NS = 16
CHUNK = 80


def _mesh():
    return plsc.VectorSubcoreMesh(core_axis_name="c", subcore_axis_name="s")


def _sc_degrees(src, dst, zeros1d):
    e = src.shape[0]
    n = zeros1d.shape[0]
    per_tile = e // (NC * NS)
    steps = per_tile // CHUNK

    @functools.partial(
        pl.kernel,
        mesh=_mesh(),
        compiler_params=pltpu.CompilerParams(needs_layout_passes=False),
        out_type=(
            jax.ShapeDtypeStruct((NC * NS * n,), jnp.float32),
            jax.ShapeDtypeStruct((NC * NS * n,), jnp.float32),
        ),
        scratch_types=[
            pltpu.VMEM((e // (NC * NS),), jnp.int32),
            pltpu.VMEM((e // (NC * NS),), jnp.int32),
            pltpu.VMEM((n,), jnp.float32),
            pltpu.VMEM((n,), jnp.float32),
        ],
    )
    def deg_kernel(src_hbm, dst_hbm, z_hbm, outa_hbm, outb_hbm,
                   idx_s, idx_d, ha, hb):
        cid = lax.axis_index("c")
        sid = lax.axis_index("s")
        wid = cid * NS + sid
        pltpu.sync_copy(z_hbm, ha)
        pltpu.sync_copy(z_hbm, hb)
        base = wid * per_tile
        ones = jnp.full((16,), 1.0, jnp.float32)

        pltpu.sync_copy(src_hbm.at[pl.ds(base, per_tile)], idx_s)
        pltpu.sync_copy(dst_hbm.at[pl.ds(base, per_tile)], idx_d)

        def body(j, carry):
            plsc.addupdate_scatter(ha, [idx_s[pl.ds(j * 16, 16)]], ones)
            plsc.addupdate_scatter(hb, [idx_d[pl.ds(j * 16, 16)]], ones)
            return carry

        lax.fori_loop(0, per_tile // 16, body, 0)
        pltpu.sync_copy(ha, outa_hbm.at[pl.ds(wid * n, n)])
        pltpu.sync_copy(hb, outb_hbm.at[pl.ds(wid * n, n)])

    return deg_kernel(src, dst, zeros1d)


def _sc_gather_scatter(table, src_flat, dst3, zeros128):
    nw, steps, chunk = dst3.shape
    pt = steps * chunk
    n, d = zeros128.shape
    s0 = (n // (8 * NS)) * 8
    tail = n - NS * s0
    assert steps % 2 == 0

    @functools.partial(
        pl.kernel,
        mesh=_mesh(),
        out_type=jax.ShapeDtypeStruct((NC, n, d), jnp.float32),
        scratch_types=[
            pltpu.VMEM((pt,), jnp.int32),
            pltpu.VMEM((steps, chunk), jnp.int32),
            pltpu.VMEM((chunk, d), jnp.float32),
            pltpu.VMEM((chunk, d), jnp.float32),
            pltpu.SemaphoreType.DMA,
            pltpu.SemaphoreType.DMA,
            pltpu.VMEM_SHARED((n, d), jnp.float32),
        ],
    )
    def gs_kernel(tab_hbm, src_hbm, dst_hbm, z_hbm, out_hbm,
                  idx_s, idx_d, r0, r1, sem0, sem1, acc):
        cid = lax.axis_index("c")
        sid = lax.axis_index("s")
        wid = cid * NS + sid

        def each_stripe(fn):
            fn(pl.ds(sid * s0, s0))
            if tail:
                @pl.when(sid == NS - 1)
                def _():
                    fn(pl.ds(NS * s0, tail))

        each_stripe(lambda st: pltpu.sync_copy(z_hbm.at[st], acc.at[st]))
        pltpu.sync_copy(src_hbm.at[pl.ds(wid * pt, pt)], idx_s)
        pltpu.sync_copy(dst_hbm.at[wid], idx_d)
        plsc.subcore_barrier()

        def start(j, rb, sem):
            pltpu.async_copy(tab_hbm.at[idx_s.at[pl.ds(j * chunk, chunk)]], rb, sem)

        def drain(j, rb, sem):
            pltpu.make_async_copy(
                tab_hbm.at[idx_s.at[pl.ds(j * chunk, chunk)]], rb, sem).wait()

        start(0, r0, sem0)

        def body(jj, carry):
            j0 = 2 * jj
            j1 = j0 + 1
            start(j1, r1, sem1)
            drain(j0, r0, sem0)
            pltpu.sync_copy(r0, acc.at[idx_d.at[j0]], add=True)
            start(j0 + 2, r0, sem0)
            drain(j1, r1, sem1)
            pltpu.sync_copy(r1, acc.at[idx_d.at[j1]], add=True)
            return carry

        lax.fori_loop(0, steps // 2 - 1, body, 0)
        start(steps - 1, r1, sem1)
        drain(steps - 2, r0, sem0)
        pltpu.sync_copy(r0, acc.at[idx_d.at[steps - 2]], add=True)
        drain(steps - 1, r1, sem1)
        pltpu.sync_copy(r1, acc.at[idx_d.at[steps - 1]], add=True)

        plsc.subcore_barrier()
        each_stripe(
            lambda st: pltpu.sync_copy(acc.at[st], out_hbm.at[cid].at[st]))

    return gs_kernel(table, src_flat, dst3, zeros128)


def _sc_edge_dots(hn, src_flat, dst_flat, steps, chunk):
    pt = steps * chunk
    e_pad = NC * NS * pt
    n8, d = hn.shape
    nsub = d // 16
    rows_per_chunk = chunk // 8
    s0s = (n8 // (8 * NS)) * 8
    tail = n8 - NS * s0s
    assert steps % 2 == 0 and chunk % 16 == 0

    @functools.partial(
        pl.kernel,
        mesh=_mesh(),
        compiler_params=pltpu.CompilerParams(needs_layout_passes=False),
        out_type=jax.ShapeDtypeStruct((e_pad // 8, 128), jnp.float32),
        scratch_types=[
            pltpu.VMEM((pt,), jnp.int32),
            pltpu.VMEM((pt,), jnp.int32),
            pltpu.VMEM((chunk, d), jnp.float32),
            pltpu.VMEM((chunk, d), jnp.float32),
            pltpu.VMEM((chunk, d), jnp.float32),
            pltpu.VMEM((chunk, d), jnp.float32),
            pltpu.VMEM((2 * rows_per_chunk, 128), jnp.float32),
            pltpu.SemaphoreType.DMA,
            pltpu.SemaphoreType.DMA,
            pltpu.VMEM_SHARED((n8, d), jnp.float32),
        ],
    )
    def dot_kernel(hn_hbm, src_hbm, dst_hbm, out_hbm,
                   idx_s, idx_d, s0b, d0b, s1b, d1b, res2, sem0, sem1, tab_s):
        cid = lax.axis_index("c")
        sid = lax.axis_index("s")
        wid = cid * NS + sid
        base = wid * pt
        row_base = wid * (pt // 8)
        st = pl.ds(sid * s0s, s0s)
        pltpu.sync_copy(hn_hbm.at[st], tab_s.at[st])
        if tail:
            @pl.when(sid == NS - 1)
            def _():
                st2 = pl.ds(NS * s0s, tail)
                pltpu.sync_copy(hn_hbm.at[st2], tab_s.at[st2])
        pltpu.sync_copy(src_hbm.at[pl.ds(base, pt)], idx_s)
        pltpu.sync_copy(dst_hbm.at[pl.ds(base, pt)], idx_d)
        plsc.subcore_barrier()

        def start(j, sb, db, sem):
            sl = pl.ds(j * chunk, chunk)
            pltpu.async_copy(tab_s.at[idx_s.at[sl]], sb, sem)
            pltpu.async_copy(tab_s.at[idx_d.at[sl]], db, sem)

        def drain(j, sb, db, sem):
            sl = pl.ds(j * chunk, chunk)
            pltpu.make_async_copy(tab_s.at[idx_s.at[sl]], sb, sem).wait()
            pltpu.make_async_copy(tab_s.at[idx_d.at[sl]], db, sem).wait()

        def compute(parity, sb, db):
            def block(g8, c2):
                row = parity * rows_per_chunk + g8
                for u in range(8):
                    ei = g8 * 8 + u
                    ta = sb[ei, pl.ds(0, 16)] * db[ei, pl.ds(0, 16)]
                    tb = sb[ei, pl.ds(16, 16)] * db[ei, pl.ds(16, 16)]
                    for k in range(2, nsub, 2):
                        ta = ta + sb[ei, pl.ds(16 * k, 16)] * db[ei, pl.ds(16 * k, 16)]
                        tb = tb + sb[ei, pl.ds(16 * (k + 1), 16)] * db[ei, pl.ds(16 * (k + 1), 16)]
                    res2[row, pl.ds(u * 16, 16)] = ta + tb
                return c2

            lax.fori_loop(0, rows_per_chunk, block, 0)

        def flush(jj):
            pltpu.sync_copy(
                res2,
                out_hbm.at[pl.ds(row_base + jj * 2 * rows_per_chunk,
                                 2 * rows_per_chunk)])

        start(0, s0b, d0b, sem0)

        def body(jj, carry):
            j0 = 2 * jj
            j1 = j0 + 1
            start(j1, s1b, d1b, sem1)
            drain(j0, s0b, d0b, sem0)
            compute(0, s0b, d0b)
            start(j0 + 2, s0b, d0b, sem0)
            drain(j1, s1b, d1b, sem1)
            compute(1, s1b, d1b)
            flush(jj)
            return carry

        lax.fori_loop(0, steps // 2 - 1, body, 0)
        start(steps - 1, s1b, d1b, sem1)
        drain(steps - 2, s0b, d0b, sem0)
        compute(0, s0b, d0b)
        drain(steps - 1, s1b, d1b, sem1)
        compute(1, s1b, d1b)
        flush(steps // 2 - 1)

    return dot_kernel(hn, src_flat, dst_flat)


def _tc_scale_src(x, hsT, hdT):
    n, d = x.shape

    def f(x_ref, hs_ref, hd_ref, o_ref, ns_ref, nd_ref):
        ns = lax.rsqrt(jnp.maximum(jnp.sum(hs_ref[...], axis=1, keepdims=True), 1.0))
        nd = lax.rsqrt(jnp.maximum(jnp.sum(hd_ref[...], axis=1, keepdims=True), 1.0))
        ns_ref[...] = ns
        nd_ref[...] = nd
        o_ref[:n, :] = x_ref[...] * ns
        o_ref[n:, :] = jnp.zeros((8, d), jnp.float32)

    return pl.pallas_call(
        f,
        out_shape=(
            jax.ShapeDtypeStruct((n + 8, d), jnp.float32),
            jax.ShapeDtypeStruct((n, 1), jnp.float32),
            jax.ShapeDtypeStruct((n, 1), jnp.float32),
        ),
    )(x, hsT, hdT)


def _tc_combine_mid(p0, p1, nd_col, ns_col, W, b):
    n, d = p0.shape

    def f(p0_ref, p1_ref, nd_ref, ns_ref, w_ref, b_ref, o_ref):
        agg = (p0_ref[...] + p1_ref[...]) * nd_ref[...]
        h = jnp.dot(agg, w_ref[...], preferred_element_type=jnp.float32) + b_ref[...]
        o_ref[:n, :] = h * ns_ref[...]
        o_ref[n:, :] = jnp.zeros((8, d), jnp.float32)

    return pl.pallas_call(
        f, out_shape=jax.ShapeDtypeStruct((n + 8, d), jnp.float32)
    )(p0, p1, nd_col, ns_col, W, b)


def _tc_combine_final(q0, q1, nd_col, W, b):
    n, d = q0.shape

    def f(q0_ref, q1_ref, nd_ref, w_ref, b_ref, o_ref):
        agg = (q0_ref[...] + q1_ref[...]) * nd_ref[...]
        h = jnp.dot(agg, w_ref[...], preferred_element_type=jnp.float32) + b_ref[...]
        nrm = jnp.sqrt(jnp.sum(h * h, axis=1, keepdims=True))
        o_ref[:n, :] = h / jnp.maximum(nrm, 1e-8)
        o_ref[n:, :] = jnp.zeros((8, d), jnp.float32)

    return pl.pallas_call(
        f, out_shape=jax.ShapeDtypeStruct((n + 8, d), jnp.float32)
    )(q0, q1, nd_col, W, b)


def _tc_pack_reduce(part_rows, sel):
    def f(p_ref, s_ref, o_ref):
        o_ref[...] = jnp.dot(p_ref[...], s_ref[...],
                             preferred_element_type=jnp.float32)

    r = part_rows.shape[0]
    return pl.pallas_call(
        f, out_shape=jax.ShapeDtypeStruct((r, 8), jnp.float32)
    )(part_rows, sel)


def kernel(x, edge_index, W1, b1, W2, b2):
    n, d = x.shape
    src = edge_index[0].astype(jnp.int32)
    dst = edge_index[1].astype(jnp.int32)
    e = src.shape[0]
    nw = NC * NS
    pt = e // nw
    ch = 96
    steps = -(-pt // ch)
    if steps % 2:
        steps += 1
    pt_pad = steps * ch
    pad = pt_pad - pt
    src2 = src.reshape(nw, pt)
    dst2 = dst.reshape(nw, pt)
    src_flat = jnp.pad(src2, ((0, 0), (0, pad)), constant_values=n).reshape(-1)
    dst_flat = jnp.pad(dst2, ((0, 0), (0, pad))).reshape(-1)
    dst3 = jnp.pad(dst2, ((0, 0), (0, pad))).reshape(nw, steps, ch)

    zeros1d = jnp.zeros((n,), jnp.float32)
    zeros128 = jnp.zeros((n, d), jnp.float32)
    b1r = b1.reshape(1, d)
    b2r = b2.reshape(1, d)

    hs_flat, hd_flat = _sc_degrees(src, dst, zeros1d)
    xs_pad, ns_col, nd_col = _tc_scale_src(
        x, hs_flat.reshape(NC * NS, n).T, hd_flat.reshape(NC * NS, n).T)

    p = _sc_gather_scatter(xs_pad, src_flat, dst3, zeros128)
    h1s_pad = _tc_combine_mid(p[0], p[1], nd_col, ns_col, W1, b1r)

    q = _sc_gather_scatter(h1s_pad, src_flat, dst3, zeros128)
    hn_pad = _tc_combine_final(q[0], q[1], nd_col, W2, b2r)

    ch_d = 32
    steps_d = -(-pt // ch_d)
    if steps_d % 2:
        steps_d += 1
    ptd_pad = steps_d * ch_d
    pad_d = ptd_pad - pt
    srcd_flat = jnp.pad(src2, ((0, 0), (0, pad_d))).reshape(-1)
    dstd_flat = jnp.pad(dst2, ((0, 0), (0, pad_d))).reshape(-1)
    part_rows = _sc_edge_dots(hn_pad, srcd_flat, dstd_flat, steps_d, ch_d)
    sel = (jnp.arange(128)[:, None] // 16 == jnp.arange(8)[None, :]).astype(
        jnp.float32)
    dots8 = _tc_pack_reduce(part_rows, sel)
    return dots8.reshape(nw, ptd_pad)[:, :pt].reshape(-1)


# speedup vs baseline: 1.3229x; 1.0005x over previous
import functools

import jax
import jax.numpy as jnp
from jax import lax
from jax.experimental import pallas as pl
from jax.experimental.pallas import tpu as pltpu
from jax.experimental.pallas import tpu_sc as plsc

NC = 2
NS = 16


def _mesh():
    return plsc.VectorSubcoreMesh(core_axis_name="c", subcore_axis_name="s")


def _sc_degrees(src, dst, zeros1d):
    e = src.shape[0]
    n = zeros1d.shape[0]
    per_tile = e // (NC * NS)

    @functools.partial(
        pl.kernel,
        mesh=_mesh(),
        compiler_params=pltpu.CompilerParams(needs_layout_passes=False),
        out_type=(
            jax.ShapeDtypeStruct((NC * NS * n,), jnp.float32),
            jax.ShapeDtypeStruct((NC * NS * n,), jnp.float32),
        ),
        scratch_types=[
            pltpu.VMEM((e // (NC * NS),), jnp.int32),
            pltpu.VMEM((e // (NC * NS),), jnp.int32),
            pltpu.VMEM((n,), jnp.float32),
            pltpu.VMEM((n,), jnp.float32),
        ],
    )
    def deg_kernel(src_hbm, dst_hbm, z_hbm, outa_hbm, outb_hbm,
                   idx_s, idx_d, ha, hb):
        cid = lax.axis_index("c")
        sid = lax.axis_index("s")
        wid = cid * NS + sid
        pltpu.sync_copy(z_hbm, ha)
        pltpu.sync_copy(z_hbm, hb)
        base = wid * per_tile
        ones = jnp.full((16,), 1.0, jnp.float32)

        pltpu.sync_copy(src_hbm.at[pl.ds(base, per_tile)], idx_s)
        pltpu.sync_copy(dst_hbm.at[pl.ds(base, per_tile)], idx_d)

        def body(j, carry):
            plsc.addupdate_scatter(ha, [idx_s[pl.ds(j * 16, 16)]], ones)
            plsc.addupdate_scatter(hb, [idx_d[pl.ds(j * 16, 16)]], ones)
            return carry

        lax.fori_loop(0, per_tile // 16, body, 0)
        pltpu.sync_copy(ha, outa_hbm.at[pl.ds(wid * n, n)])
        pltpu.sync_copy(hb, outb_hbm.at[pl.ds(wid * n, n)])

    return deg_kernel(src, dst, zeros1d)


def _sc_gather_scatter(table, src_flat, dst3, zeros128):
    nw, steps, chunk = dst3.shape
    pt = steps * chunk
    n, d = zeros128.shape
    s0 = (n // (8 * NS)) * 8
    tail = n - NS * s0
    assert steps % 2 == 0

    @functools.partial(
        pl.kernel,
        mesh=_mesh(),
        out_type=jax.ShapeDtypeStruct((NC, n, d), jnp.float32),
        scratch_types=[
            pltpu.VMEM((pt,), jnp.int32),
            pltpu.VMEM((steps, chunk), jnp.int32),
            pltpu.VMEM((chunk, d), jnp.float32),
            pltpu.VMEM((chunk, d), jnp.float32),
            pltpu.SemaphoreType.DMA,
            pltpu.SemaphoreType.DMA,
            pltpu.VMEM_SHARED((n, d), jnp.float32),
        ],
    )
    def gs_kernel(tab_hbm, src_hbm, dst_hbm, z_hbm, out_hbm,
                  idx_s, idx_d, r0, r1, sem0, sem1, acc):
        cid = lax.axis_index("c")
        sid = lax.axis_index("s")
        wid = cid * NS + sid

        def each_stripe(fn):
            fn(pl.ds(sid * s0, s0))
            if tail:
                @pl.when(sid == NS - 1)
                def _():
                    fn(pl.ds(NS * s0, tail))

        each_stripe(lambda st: pltpu.sync_copy(z_hbm.at[st], acc.at[st]))
        pltpu.sync_copy(src_hbm.at[pl.ds(wid * pt, pt)], idx_s)
        pltpu.sync_copy(dst_hbm.at[wid], idx_d)
        plsc.subcore_barrier()

        def start(j, rb, sem):
            pltpu.async_copy(tab_hbm.at[idx_s.at[pl.ds(j * chunk, chunk)]], rb, sem)

        def drain(j, rb, sem):
            pltpu.make_async_copy(
                tab_hbm.at[idx_s.at[pl.ds(j * chunk, chunk)]], rb, sem).wait()

        start(0, r0, sem0)

        def body(jj, carry):
            j0 = 2 * jj
            j1 = j0 + 1
            start(j1, r1, sem1)
            drain(j0, r0, sem0)
            pltpu.sync_copy(r0, acc.at[idx_d.at[j0]], add=True)
            start(j0 + 2, r0, sem0)
            drain(j1, r1, sem1)
            pltpu.sync_copy(r1, acc.at[idx_d.at[j1]], add=True)
            return carry

        lax.fori_loop(0, steps // 2 - 1, body, 0)
        start(steps - 1, r1, sem1)
        drain(steps - 2, r0, sem0)
        pltpu.sync_copy(r0, acc.at[idx_d.at[steps - 2]], add=True)
        drain(steps - 1, r1, sem1)
        pltpu.sync_copy(r1, acc.at[idx_d.at[steps - 1]], add=True)

        plsc.subcore_barrier()
        each_stripe(
            lambda st: pltpu.sync_copy(acc.at[st], out_hbm.at[cid].at[st]))

    return gs_kernel(table, src_flat, dst3, zeros128)


def _sc_edge_dots(hn, src_flat, dst_flat, steps, chunk):
    pt = steps * chunk
    e_pad = NC * NS * pt
    n8, d = hn.shape
    nsub = d // 16
    rows_per_chunk = chunk // 8
    s0s = (n8 // (8 * NS)) * 8
    tail = n8 - NS * s0s
    assert steps % 2 == 0 and chunk % 16 == 0

    @functools.partial(
        pl.kernel,
        mesh=_mesh(),
        compiler_params=pltpu.CompilerParams(needs_layout_passes=False),
        out_type=jax.ShapeDtypeStruct((e_pad // 8, 128), jnp.float32),
        scratch_types=[
            pltpu.VMEM((pt,), jnp.int32),
            pltpu.VMEM((pt,), jnp.int32),
            pltpu.VMEM((chunk, d), jnp.float32),
            pltpu.VMEM((chunk, d), jnp.float32),
            pltpu.VMEM((chunk, d), jnp.float32),
            pltpu.VMEM((chunk, d), jnp.float32),
            pltpu.VMEM((2 * rows_per_chunk, 128), jnp.float32),
            pltpu.SemaphoreType.DMA,
            pltpu.SemaphoreType.DMA,
            pltpu.VMEM_SHARED((n8, d), jnp.float32),
        ],
    )
    def dot_kernel(hn_hbm, src_hbm, dst_hbm, out_hbm,
                   idx_s, idx_d, s0b, d0b, s1b, d1b, res2, sem0, sem1, tab_s):
        cid = lax.axis_index("c")
        sid = lax.axis_index("s")
        wid = cid * NS + sid
        base = wid * pt
        row_base = wid * (pt // 8)
        st = pl.ds(sid * s0s, s0s)
        pltpu.sync_copy(hn_hbm.at[st], tab_s.at[st])
        if tail:
            @pl.when(sid == NS - 1)
            def _():
                st2 = pl.ds(NS * s0s, tail)
                pltpu.sync_copy(hn_hbm.at[st2], tab_s.at[st2])
        pltpu.sync_copy(src_hbm.at[pl.ds(base, pt)], idx_s)
        pltpu.sync_copy(dst_hbm.at[pl.ds(base, pt)], idx_d)
        plsc.subcore_barrier()

        def start(j, sb, db, sem):
            sl = pl.ds(j * chunk, chunk)
            pltpu.async_copy(tab_s.at[idx_s.at[sl]], sb, sem)
            pltpu.async_copy(tab_s.at[idx_d.at[sl]], db, sem)

        def drain(j, sb, db, sem):
            sl = pl.ds(j * chunk, chunk)
            pltpu.make_async_copy(tab_s.at[idx_s.at[sl]], sb, sem).wait()
            pltpu.make_async_copy(tab_s.at[idx_d.at[sl]], db, sem).wait()

        def compute(parity, sb, db):
            def block(g8, c2):
                row = parity * rows_per_chunk + g8
                for u in range(8):
                    ei = g8 * 8 + u
                    ta = sb[ei, pl.ds(0, 16)] * db[ei, pl.ds(0, 16)]
                    tb = sb[ei, pl.ds(16, 16)] * db[ei, pl.ds(16, 16)]
                    for k in range(2, nsub, 2):
                        ta = ta + sb[ei, pl.ds(16 * k, 16)] * db[ei, pl.ds(16 * k, 16)]
                        tb = tb + sb[ei, pl.ds(16 * (k + 1), 16)] * db[ei, pl.ds(16 * (k + 1), 16)]
                    res2[row, pl.ds(u * 16, 16)] = ta + tb
                return c2

            lax.fori_loop(0, rows_per_chunk, block, 0)

        def flush(jj):
            pltpu.sync_copy(
                res2,
                out_hbm.at[pl.ds(row_base + jj * 2 * rows_per_chunk,
                                 2 * rows_per_chunk)])

        start(0, s0b, d0b, sem0)

        def body(jj, carry):
            j0 = 2 * jj
            j1 = j0 + 1
            start(j1, s1b, d1b, sem1)
            drain(j0, s0b, d0b, sem0)
            compute(0, s0b, d0b)
            start(j0 + 2, s0b, d0b, sem0)
            drain(j1, s1b, d1b, sem1)
            compute(1, s1b, d1b)
            flush(jj)
            return carry

        lax.fori_loop(0, steps // 2 - 1, body, 0)
        start(steps - 1, s1b, d1b, sem1)
        drain(steps - 2, s0b, d0b, sem0)
        compute(0, s0b, d0b)
        drain(steps - 1, s1b, d1b, sem1)
        compute(1, s1b, d1b)
        flush(steps // 2 - 1)

    return dot_kernel(hn, src_flat, dst_flat)


def _tc_scale_src(x, hsT, hdT):
    n, d = x.shape

    def f(x_ref, hs_ref, hd_ref, o_ref, ns_ref, nd_ref):
        ns = lax.rsqrt(jnp.maximum(jnp.sum(hs_ref[...], axis=1, keepdims=True), 1.0))
        nd = lax.rsqrt(jnp.maximum(jnp.sum(hd_ref[...], axis=1, keepdims=True), 1.0))
        ns_ref[...] = ns
        nd_ref[...] = nd
        o_ref[:n, :] = x_ref[...] * ns
        o_ref[n:, :] = jnp.zeros((8, d), jnp.float32)

    return pl.pallas_call(
        f,
        out_shape=(
            jax.ShapeDtypeStruct((n + 8, d), jnp.float32),
            jax.ShapeDtypeStruct((n, 1), jnp.float32),
            jax.ShapeDtypeStruct((n, 1), jnp.float32),
        ),
    )(x, hsT, hdT)


def _tc_combine_mid(p0, p1, nd_col, ns_col, W, b):
    n, d = p0.shape

    def f(p0_ref, p1_ref, nd_ref, ns_ref, w_ref, b_ref, o_ref):
        agg = (p0_ref[...] + p1_ref[...]) * nd_ref[...]
        h = jnp.dot(agg, w_ref[...], preferred_element_type=jnp.float32) + b_ref[...]
        o_ref[:n, :] = h * ns_ref[...]
        o_ref[n:, :] = jnp.zeros((8, d), jnp.float32)

    return pl.pallas_call(
        f, out_shape=jax.ShapeDtypeStruct((n + 8, d), jnp.float32)
    )(p0, p1, nd_col, ns_col, W, b)


def _tc_combine_final(q0, q1, nd_col, W, b):
    n, d = q0.shape

    def f(q0_ref, q1_ref, nd_ref, w_ref, b_ref, o_ref):
        agg = (q0_ref[...] + q1_ref[...]) * nd_ref[...]
        h = jnp.dot(agg, w_ref[...], preferred_element_type=jnp.float32) + b_ref[...]
        nrm = jnp.sqrt(jnp.sum(h * h, axis=1, keepdims=True))
        o_ref[:n, :] = h / jnp.maximum(nrm, 1e-8)
        o_ref[n:, :] = jnp.zeros((8, d), jnp.float32)

    return pl.pallas_call(
        f, out_shape=jax.ShapeDtypeStruct((n + 8, d), jnp.float32)
    )(q0, q1, nd_col, W, b)


def _tc_pack_reduce(part_rows, sel):
    def f(p_ref, s_ref, o_ref):
        o_ref[...] = jnp.dot(p_ref[...], s_ref[...],
                             preferred_element_type=jnp.float32)

    r = part_rows.shape[0]
    return pl.pallas_call(
        f, out_shape=jax.ShapeDtypeStruct((r, 8), jnp.float32)
    )(part_rows, sel)


def kernel(x, edge_index, W1, b1, W2, b2):
    n, d = x.shape
    src = edge_index[0].astype(jnp.int32)
    dst = edge_index[1].astype(jnp.int32)
    e = src.shape[0]
    nw = NC * NS
    pt = e // nw
    ch = 96
    steps = -(-pt // ch)
    if steps % 2:
        steps += 1
    pt_pad = steps * ch
    pad = pt_pad - pt
    src2 = src.reshape(nw, pt)
    dst2 = dst.reshape(nw, pt)
    src_flat = jnp.pad(src2, ((0, 0), (0, pad)), constant_values=n).reshape(-1)
    dst_flat = jnp.pad(dst2, ((0, 0), (0, pad))).reshape(-1)
    dst3 = jnp.pad(dst2, ((0, 0), (0, pad))).reshape(nw, steps, ch)

    zeros1d = jnp.zeros((n,), jnp.float32)
    zeros128 = jnp.zeros((n, d), jnp.float32)
    b1r = b1.reshape(1, d)
    b2r = b2.reshape(1, d)

    hs_flat, hd_flat = _sc_degrees(src, dst, zeros1d)
    xs_pad, ns_col, nd_col = _tc_scale_src(
        x, hs_flat.reshape(NC * NS, n).T, hd_flat.reshape(NC * NS, n).T)

    p = _sc_gather_scatter(xs_pad, src_flat, dst3, zeros128)
    h1s_pad = _tc_combine_mid(p[0], p[1], nd_col, ns_col, W1, b1r)

    q = _sc_gather_scatter(h1s_pad, src_flat, dst3, zeros128)
    hn_pad = _tc_combine_final(q[0], q[1], nd_col, W2, b2r)

    ch_d = 32
    steps_d = -(-pt // ch_d)
    if steps_d % 2:
        steps_d += 1
    ptd_pad = steps_d * ch_d
    pad_d = ptd_pad - pt
    srcd_flat = jnp.pad(src2, ((0, 0), (0, pad_d))).reshape(-1)
    dstd_flat = jnp.pad(dst2, ((0, 0), (0, pad_d))).reshape(-1)
    part_rows = _sc_edge_dots(hn_pad, srcd_flat, dstd_flat, steps_d, ch_d)
    sel = (jnp.arange(128)[:, None] // 16 == jnp.arange(8)[None, :]).astype(
        jnp.float32)
    dots8 = _tc_pack_reduce(part_rows, sel)
    return dots8.reshape(nw, ptd_pad)[:, :pt].reshape(-1)
